# Initial kernel scaffold; baseline (speedup 1.0000x reference)
import jax
import jax.numpy as jnp
from jax.experimental import pallas as pl


def kernel(extended_coord, extended_atype, nlist, type_embedding, W_e_init, b_e_init, W_node, W_edge):
    raise NotImplementedError("write your pallas kernel here")


# speedup vs baseline: 6.5065x; 6.5065x over previous
import functools

import jax
import jax.numpy as jnp
from jax import lax
from jax.experimental import pallas as pl
from jax.experimental.pallas import tpu as pltpu
from jax.experimental.pallas import tpu_sc as plsc

_NTYPES = 4
_N_DIM = 128
_E_DIM = 64
_AXIS = 4
_NLAYERS = 3
_RCUT = 6.0
_RCUT_SMTH = 5.0
_EPS = 1e-6

_BA = 80
_CK = 80
_NBUF = 5


@functools.lru_cache(maxsize=None)
def _make_sc_gather(V, D, B):
    info = plsc.get_sparse_core_info()
    nc, ns = info.num_cores, info.num_subcores
    nw = nc * ns
    per_w = B // nw
    assert per_w * nw == B and per_w % _CK == 0
    nch = per_w // _CK
    nbuf = _NBUF
    assert nch % nbuf == 0
    rounds = nch // nbuf
    mesh = plsc.VectorSubcoreMesh(core_axis_name="c", subcore_axis_name="s")

    @functools.partial(
        pl.kernel,
        mesh=mesh,
        compiler_params=pltpu.CompilerParams(use_tc_tiling_on_sc=False),
        out_type=jax.ShapeDtypeStruct((B, D), jnp.float32),
        scratch_types=(
            [pltpu.VMEM((per_w,), jnp.int32)]
            + [pltpu.VMEM((_CK, D), jnp.float32) for _ in range(nbuf)]
            + [pltpu.SemaphoreType.DMA for _ in range(nbuf)]
        ),
    )
    def gather(table_hbm, idx_hbm, out_hbm, idx_v, *rest):
        bufs = rest[:nbuf]
        sems = rest[nbuf:]
        wid = lax.axis_index("s") * nc + lax.axis_index("c")
        row0 = wid * per_w
        pltpu.sync_copy(idx_hbm.at[pl.ds(row0, per_w)], idx_v)

        @pl.loop(0, rounds)
        def _(g):
            base = g * nbuf
            ds = [
                pltpu.async_copy(
                    table_hbm.at[idx_v.at[pl.ds((base + b) * _CK, _CK)]],
                    bufs[b], sems[b])
                for b in range(nbuf)
            ]
            for b in range(nbuf):
                ds[b].wait()
                pltpu.sync_copy(
                    bufs[b], out_hbm.at[pl.ds(row0 + (base + b) * _CK, _CK)]
                )

    return gather


def _sc_gather(table, idx):
    V, D = table.shape
    B = idx.shape[0]
    return _make_sc_gather(V, D, B)(table, idx)


def _silu(x):
    return x * (1.0 / (1.0 + jnp.exp(-x)))


def _tc_init_body(cnb_ref, coord_ref, atype_ref, temb_ref, we_ref, be_ref,
                  w3e_ref, env0_ref, env1_ref, env2_ref, env3_ref, sw_ref,
                  edge_ref, node_ref, p_ref):
    cn = cnb_ref[...]
    cc = coord_ref[...]
    dx = cn[:, :, 0:1] - cc[:, 0:1][:, None, :]
    dy = cn[:, :, 1:2] - cc[:, 1:2][:, None, :]
    dz = cn[:, :, 2:3] - cc[:, 2:3][:, None, :]
    r2 = dx * dx + dy * dy + dz * dz + 1e-12
    dist = jnp.sqrt(r2)
    uu = jnp.clip((dist - _RCUT_SMTH) / (_RCUT - _RCUT_SMTH), 0.0, 1.0)
    sw = uu ** 3 * (-6.0 * uu ** 2 + 15.0 * uu - 10.0) + 1.0
    inv = sw / (dist + _EPS)
    inv2 = inv / (dist + _EPS)
    e0, e1, e2, e3 = inv, inv2 * dx, inv2 * dy, inv2 * dz
    env0_ref[...] = e0
    env1_ref[...] = e1
    env2_ref[...] = e2
    env3_ref[...] = e3
    sw_ref[...] = sw

    we = we_ref[...]
    acc = jnp.broadcast_to(be_ref[...][None], e0.shape[:2] + (_E_DIM,))
    acc = acc + e0 * we[0:1, :][None]
    acc = acc + e1 * we[1:2, :][None]
    acc = acc + e2 * we[2:3, :][None]
    acc = acc + e3 * we[3:4, :][None]
    edge_ref[...] = _silu(acc)

    at = atype_ref[...]
    ids = lax.broadcasted_iota(jnp.int32, (at.shape[0], _NTYPES), 1)
    onehot = (ids == at).astype(jnp.float32)
    node0 = jnp.dot(onehot, temb_ref[...], preferred_element_type=jnp.float32)
    node_ref[...] = node0
    p_ref[...] = jnp.dot(node0, w3e_ref[...], preferred_element_type=jnp.float32)


def _tc_layer_body(last, edge_ref, pg_ref, env0_ref, env1_ref, env2_ref,
                   env3_ref, sw_ref, node_ref, wn1_ref, wn2_ref, wn3_ref,
                   w1e_ref, w2e_ref, w3n_ref, edge1_ref, node1_ref, *outs):
    ba = edge_ref.shape[0]
    edge = edge_ref[...]
    sw = sw_ref[...]
    g2 = edge * sw
    inv_n = 1.0 / 32.0
    ea = jnp.sum(g2, axis=1) * inv_n
    grs = [jnp.sum(env_ref[...] * g2, axis=1) * inv_n
           for env_ref in (env0_ref, env1_ref, env2_ref, env3_ref)]
    gparts = []
    for a in range(_AXIS):
        acc = grs[0] * grs[0][:, a:a + 1]
        for d in range(1, 4):
            acc = acc + grs[d] * grs[d][:, a:a + 1]
        gparts.append(acc)
    grrg = jnp.concatenate(gparts, axis=1)

    node = node_ref[...]
    pre = jnp.dot(node, wn1_ref[...], preferred_element_type=jnp.float32)
    pre = pre + jnp.dot(ea, wn2_ref[...], preferred_element_type=jnp.float32)
    pre = pre + jnp.dot(grrg, wn3_ref[...], preferred_element_type=jnp.float32)
    node1 = node + _silu(pre)
    node1_ref[...] = node1

    e2d = edge.reshape(ba * 32, _E_DIM)
    e1 = jnp.dot(e2d, w1e_ref[...], preferred_element_type=jnp.float32)
    e1 = e1.reshape(ba, 32, _E_DIM)
    q = jnp.dot(node1, w2e_ref[...], preferred_element_type=jnp.float32)
    epre = e1 + q.reshape(ba, 1, _E_DIM) + pg_ref[...]
    edge1 = edge + _silu(epre)
    edge1_ref[...] = edge1

    if last:
        rot1_ref, rot2_ref, rot3_ref = outs
        g2f = edge1 * sw
        rot1_ref[...] = jnp.sum(env1_ref[...] * g2f, axis=1) * inv_n
        rot2_ref[...] = jnp.sum(env2_ref[...] * g2f, axis=1) * inv_n
        rot3_ref[...] = jnp.sum(env3_ref[...] * g2f, axis=1) * inv_n
    else:
        (p_ref,) = outs
        p_ref[...] = jnp.dot(node1, w3n_ref[...],
                             preferred_element_type=jnp.float32)


def _full(shape):
    return pl.BlockSpec(shape, lambda i: (0,) * len(shape))


def _tc_init(cnb3, coordp, atype2, temb, we, be2, w3e0, nall, nnei):
    nb = nall // _BA
    bs_sc = pl.BlockSpec((_BA, nnei, 1), lambda i: (i, 0, 0))
    out_shapes = (
        [jax.ShapeDtypeStruct((nall, nnei, 1), jnp.float32)] * 5
        + [jax.ShapeDtypeStruct((nall, nnei, _E_DIM), jnp.float32),
           jax.ShapeDtypeStruct((nall, _N_DIM), jnp.float32),
           jax.ShapeDtypeStruct((nall, _E_DIM), jnp.float32)]
    )
    out_specs = (
        [bs_sc] * 5
        + [pl.BlockSpec((_BA, nnei, _E_DIM), lambda i: (i, 0, 0)),
           pl.BlockSpec((_BA, _N_DIM), lambda i: (i, 0)),
           pl.BlockSpec((_BA, _E_DIM), lambda i: (i, 0))]
    )
    return pl.pallas_call(
        _tc_init_body,
        grid=(nb,),
        in_specs=[
            pl.BlockSpec((_BA, nnei, 16), lambda i: (i, 0, 0)),
            pl.BlockSpec((_BA, 16), lambda i: (i, 0)),
            pl.BlockSpec((_BA, 1), lambda i: (i, 0)),
            _full((_NTYPES, _N_DIM)),
            _full((4, _E_DIM)),
            _full((1, _E_DIM)),
            _full((_N_DIM, _E_DIM)),
        ],
        out_specs=tuple(out_specs),
        out_shape=tuple(out_shapes),
    )(cnb3, coordp, atype2, temb, we, be2, w3e0)


def _tc_layer(last, edge, pg3, envs, sw, node, wn1, wn2, wn3, w1e, w2e, w3n,
              nall, nnei):
    nb = nall // _BA
    bs_sc = pl.BlockSpec((_BA, nnei, 1), lambda i: (i, 0, 0))
    bs_e = pl.BlockSpec((_BA, nnei, _E_DIM), lambda i: (i, 0, 0))
    bs_n = pl.BlockSpec((_BA, _N_DIM), lambda i: (i, 0))
    bs_p = pl.BlockSpec((_BA, _E_DIM), lambda i: (i, 0))
    out_shapes = [jax.ShapeDtypeStruct((nall, nnei, _E_DIM), jnp.float32),
                  jax.ShapeDtypeStruct((nall, _N_DIM), jnp.float32)]
    out_specs = [bs_e, bs_n]
    if last:
        out_shapes += [jax.ShapeDtypeStruct((nall, _E_DIM), jnp.float32)] * 3
        out_specs += [bs_p] * 3
    else:
        out_shapes += [jax.ShapeDtypeStruct((nall, _E_DIM), jnp.float32)]
        out_specs += [bs_p]
    return pl.pallas_call(
        functools.partial(_tc_layer_body, last),
        grid=(nb,),
        in_specs=[
            bs_e, bs_e, bs_sc, bs_sc, bs_sc, bs_sc, bs_sc, bs_n,
            _full((_N_DIM, _N_DIM)),
            _full((_E_DIM, _N_DIM)),
            _full((_E_DIM * _AXIS, _N_DIM)),
            _full((_E_DIM, _E_DIM)),
            _full((_N_DIM, _E_DIM)),
            _full((_N_DIM, _E_DIM)),
        ],
        out_specs=tuple(out_specs),
        out_shape=tuple(out_shapes),
    )(edge, pg3, envs[0], envs[1], envs[2], envs[3], sw, node,
      wn1, wn2, wn3, w1e, w2e, w3n)


def _one_frame(coord, atype, nl, type_embedding, W_e_init, b_e_init, W_node,
               W_edge):
    nall, nnei = nl.shape
    nl = nl.astype(jnp.int32)
    idx_flat = nl.reshape(-1)

    coordp = jnp.concatenate(
        [coord.astype(jnp.float32),
         jnp.zeros((nall, 13), jnp.float32)], axis=1)
    cnb = _sc_gather(coordp, idx_flat)
    cnb3 = cnb.reshape(nall, nnei, 16)
    atype2 = atype.astype(jnp.int32).reshape(nall, 1)

    w3e = [W_edge[l][_E_DIM + _N_DIM:] for l in range(_NLAYERS)]
    be2 = b_e_init.reshape(1, _E_DIM)

    (env0, env1, env2, env3, sw, edge, node, p) = _tc_init(
        cnb3, coordp, atype2, type_embedding, W_e_init, be2, w3e[0],
        nall, nnei)
    envs = (env0, env1, env2, env3)

    rot = None
    for l in range(_NLAYERS):
        pg3 = _sc_gather(p, idx_flat).reshape(nall, nnei, _E_DIM)
        wn = W_node[l]
        wn1, wn2 = wn[:_N_DIM], wn[_N_DIM:_N_DIM + _E_DIM]
        wn3 = (wn[_N_DIM + _E_DIM:]
               .reshape(_E_DIM, _AXIS, _N_DIM)
               .transpose(1, 0, 2)
               .reshape(_E_DIM * _AXIS, _N_DIM))
        w1e = W_edge[l][:_E_DIM]
        w2e = W_edge[l][_E_DIM:_E_DIM + _N_DIM]
        last = l == _NLAYERS - 1
        w3n = w3e[l + 1] if not last else w3e[l]
        res = _tc_layer(last, edge, pg3, envs, sw, node,
                        wn1, wn2, wn3, w1e, w2e, w3n, nall, nnei)
        if last:
            edge, node, r1, r2, r3 = res
            rot = jnp.stack([r1, r2, r3], axis=1)
        else:
            edge, node, p = res

    h2 = jnp.concatenate([env1, env2, env3], axis=-1)
    return node, edge, h2, rot, sw.reshape(nall, nnei)


def kernel(extended_coord, extended_atype, nlist, type_embedding, W_e_init,
           b_e_init, W_node, W_edge):
    nf = extended_coord.shape[0]
    outs = [
        _one_frame(extended_coord[f], extended_atype[f], nlist[f],
                   type_embedding, W_e_init, b_e_init, W_node, W_edge)
        for f in range(nf)
    ]
    return tuple(jnp.stack([o[i] for o in outs], axis=0) for i in range(5))


# speedup vs baseline: 9.3945x; 1.4439x over previous
import functools

import jax
import jax.numpy as jnp
from jax import lax
from jax.experimental import pallas as pl
from jax.experimental.pallas import tpu as pltpu
from jax.experimental.pallas import tpu_sc as plsc

_NTYPES = 4
_N_DIM = 128
_E_DIM = 64
_AXIS = 4
_NLAYERS = 3
_RCUT = 6.0
_RCUT_SMTH = 5.0
_EPS = 1e-6

_BA = 80
_BE = _BA * 32
_CH = 128
_NCH = _BE // _CH
_CK = 80
_NBUF = 5


@functools.lru_cache(maxsize=None)
def _make_sc_gather(V, D, B):
    info = plsc.get_sparse_core_info()
    nc, ns = info.num_cores, info.num_subcores
    nw = nc * ns
    per_w = B // nw
    assert per_w * nw == B and per_w % _CK == 0
    nch = per_w // _CK
    nbuf = _NBUF
    assert nch % nbuf == 0
    rounds = nch // nbuf
    mesh = plsc.VectorSubcoreMesh(core_axis_name="c", subcore_axis_name="s")

    @functools.partial(
        pl.kernel,
        mesh=mesh,
        compiler_params=pltpu.CompilerParams(use_tc_tiling_on_sc=False),
        out_type=jax.ShapeDtypeStruct((B, D), jnp.float32),
        scratch_types=(
            [pltpu.VMEM((per_w,), jnp.int32)]
            + [pltpu.VMEM((_CK, D), jnp.float32) for _ in range(nbuf)]
            + [pltpu.SemaphoreType.DMA for _ in range(nbuf)]
        ),
    )
    def gather(table_hbm, idx_hbm, out_hbm, idx_v, *rest):
        bufs = rest[:nbuf]
        sems = rest[nbuf:]
        wid = lax.axis_index("s") * nc + lax.axis_index("c")
        row0 = wid * per_w
        pltpu.sync_copy(idx_hbm.at[pl.ds(row0, per_w)], idx_v)

        @pl.loop(0, rounds)
        def _(g):
            base = g * nbuf
            ds = [
                pltpu.async_copy(
                    table_hbm.at[idx_v.at[pl.ds((base + b) * _CK, _CK)]],
                    bufs[b], sems[b])
                for b in range(nbuf)
            ]
            for b in range(nbuf):
                ds[b].wait()
                pltpu.sync_copy(
                    bufs[b], out_hbm.at[pl.ds(row0 + (base + b) * _CK, _CK)]
                )

    return gather


def _sc_gather(table, idx):
    V, D = table.shape
    B = idx.shape[0]
    return _make_sc_gather(V, D, B)(table, idx)


def _silu(x):
    return x * (1.0 / (1.0 + jnp.exp(-x)))


def _band_mask(nrows):
    row = lax.broadcasted_iota(jnp.int32, (nrows, _CH), 0)
    lane = lax.broadcasted_iota(jnp.int32, (nrows, _CH), 1)
    return (lane // 32) == (row % 4)


def _chunk_reduce(fields, edge_flat):
    nb = len(fields)
    mask = _band_mask(4 * nb)
    outs = []
    for c in range(_NCH):
        fc = jnp.stack([f[c * _CH:(c + 1) * _CH] for f in fields])
        fcr = jnp.repeat(fc, 4, axis=0)
        wc = jnp.where(mask, fcr, 0.0)
        outs.append(jnp.dot(wc, edge_flat[c * _CH:(c + 1) * _CH, :],
                            preferred_element_type=jnp.float32))
    r = jnp.stack(outs)
    return [r[:, 4 * b:4 * b + 4, :].reshape(_BA, _E_DIM) for b in range(nb)]


def _tc_init_body(cnb_ref, coordp_ref, atype_ref, temb_ref, we_ref, be_ref,
                  w3e_ref, fld_ref, e1_ref, e2_ref, e3_ref, edge_ref,
                  node_ref, p_ref):
    cn = cnb_ref[...]
    cp = coordp_ref[...]
    dx = cn[:, :, 0] - cp[:, 0:1]
    dy = cn[:, :, 1] - cp[:, 1:2]
    dz = cn[:, :, 2] - cp[:, 2:3]
    r2 = dx * dx + dy * dy + dz * dz + 1e-12
    dist = jnp.sqrt(r2)
    uu = jnp.clip((dist - _RCUT_SMTH) / (_RCUT - _RCUT_SMTH), 0.0, 1.0)
    sw = uu ** 3 * (-6.0 * uu ** 2 + 15.0 * uu - 10.0) + 1.0
    inv = sw / (dist + _EPS)
    inv2 = inv / (dist + _EPS)
    e1 = inv2 * dx
    e2 = inv2 * dy
    e3 = inv2 * dz
    e1_ref[...] = e1
    e2_ref[...] = e2
    e3_ref[...] = e3

    swf = sw.reshape(_BE)
    f0 = inv.reshape(_BE)
    f1 = e1.reshape(_BE)
    f2 = e2.reshape(_BE)
    f3 = e3.reshape(_BE)
    z = jnp.zeros((1, _BE), jnp.float32)
    fld = jnp.concatenate(
        [swf.reshape(1, _BE), (f0 * swf).reshape(1, _BE),
         (f1 * swf).reshape(1, _BE), (f2 * swf).reshape(1, _BE),
         (f3 * swf).reshape(1, _BE), z, z, z], axis=0)
    fld_ref[...] = fld.reshape(1, 8, _BE)

    wt = jnp.concatenate(
        [we_ref[...].T, be_ref[...].reshape(_E_DIM, 1)], axis=1)
    ones = jnp.ones((_CH,), jnp.float32)
    for c in range(_NCH):
        s = jnp.stack([f0[c * _CH:(c + 1) * _CH], f1[c * _CH:(c + 1) * _CH],
                       f2[c * _CH:(c + 1) * _CH], f3[c * _CH:(c + 1) * _CH],
                       ones])
        t = jnp.dot(wt, s, preferred_element_type=jnp.float32)
        edge_ref[pl.ds(c * _CH, _CH), :] = _silu(t).T

    at = atype_ref[...]
    ids = lax.broadcasted_iota(jnp.int32, (at.shape[0], _NTYPES), 1)
    onehot = (ids == at).astype(jnp.float32)
    node0 = jnp.dot(onehot, temb_ref[...], preferred_element_type=jnp.float32)
    node_ref[...] = node0
    p_ref[...] = jnp.dot(node0, w3e_ref[...], preferred_element_type=jnp.float32)


def _tc_layer_body(last, edge_ref, pg_ref, fld_ref, node_ref, wn1_ref,
                   wn2_ref, wn3_ref, w1e_ref, w2e_ref, w3n_ref,
                   edge1_ref, node1_ref, *outs):
    edge = edge_ref[...]
    fld = fld_ref[...]
    fields = [fld[0, k, :] for k in range(5)]
    inv_n = 1.0 / 32.0
    red = _chunk_reduce(fields, edge)
    ea = red[0] * inv_n
    grs = [red[a + 1] * inv_n for a in range(_AXIS)]

    gparts = []
    for a in range(_AXIS):
        sel = (lax.broadcasted_iota(jnp.int32, (_E_DIM, _E_DIM), 0) == a
               ).astype(jnp.float32)
        acc = None
        for d in range(_AXIS):
            col = jnp.dot(grs[d], sel, preferred_element_type=jnp.float32)
            t = col * grs[d]
            acc = t if acc is None else acc + t
        gparts.append(acc)
    grrg = jnp.concatenate(gparts, axis=1)

    node = node_ref[...]
    pre = jnp.dot(node, wn1_ref[...], preferred_element_type=jnp.float32)
    pre = pre + jnp.dot(ea, wn2_ref[...], preferred_element_type=jnp.float32)
    pre = pre + jnp.dot(grrg, wn3_ref[...], preferred_element_type=jnp.float32)
    node1 = node + _silu(pre)
    node1_ref[...] = node1

    e1m = jnp.dot(edge, w1e_ref[...], preferred_element_type=jnp.float32)
    q = jnp.dot(node1, w2e_ref[...], preferred_element_type=jnp.float32)
    qexp = jnp.repeat(q, 32, axis=0)
    epre = e1m + qexp + pg_ref[...]
    edge1 = edge + _silu(epre)

    if last:
        rot1_ref, rot2_ref, rot3_ref = outs
        edge1_ref[...] = edge1.reshape(_BA, 32, _E_DIM)
        rot = _chunk_reduce(fields[2:], edge1)
        rot1_ref[...] = rot[0] * inv_n
        rot2_ref[...] = rot[1] * inv_n
        rot3_ref[...] = rot[2] * inv_n
    else:
        edge1_ref[...] = edge1
        (p_ref,) = outs
        p_ref[...] = jnp.dot(node1, w3n_ref[...],
                             preferred_element_type=jnp.float32)


def _full(shape):
    return pl.BlockSpec(shape, lambda i: (0,) * len(shape))


def _tc_init(cnb3, coordp, atype2, temb, we, be2, w3e0, nall, nnei):
    nb = nall // _BA
    bs_f = pl.BlockSpec((1, 8, _BE), lambda i: (i, 0, 0))
    bs_s = pl.BlockSpec((_BA, nnei), lambda i: (i, 0))
    out_shapes = (
        [jax.ShapeDtypeStruct((nb, 8, _BE), jnp.float32)]
        + [jax.ShapeDtypeStruct((nall, nnei), jnp.float32)] * 3
        + [jax.ShapeDtypeStruct((nall * nnei, _E_DIM), jnp.float32),
           jax.ShapeDtypeStruct((nall, _N_DIM), jnp.float32),
           jax.ShapeDtypeStruct((nall, _E_DIM), jnp.float32)]
    )
    out_specs = (
        [bs_f]
        + [bs_s] * 3
        + [pl.BlockSpec((_BE, _E_DIM), lambda i: (i, 0)),
           pl.BlockSpec((_BA, _N_DIM), lambda i: (i, 0)),
           pl.BlockSpec((_BA, _E_DIM), lambda i: (i, 0))]
    )
    return pl.pallas_call(
        _tc_init_body,
        grid=(nb,),
        in_specs=[
            pl.BlockSpec((_BA, nnei, 16), lambda i: (i, 0, 0)),
            pl.BlockSpec((_BA, 16), lambda i: (i, 0)),
            pl.BlockSpec((_BA, 1), lambda i: (i, 0)),
            _full((_NTYPES, _N_DIM)),
            _full((4, _E_DIM)),
            _full((1, _E_DIM)),
            _full((_N_DIM, _E_DIM)),
        ],
        out_specs=tuple(out_specs),
        out_shape=tuple(out_shapes),
    )(cnb3, coordp, atype2, temb, we, be2, w3e0)


def _tc_layer(last, edge, pg, fld, node, wn1, wn2, wn3, w1e, w2e, w3n,
              nall, nnei):
    nb = nall // _BA
    bs_f = pl.BlockSpec((1, 8, _BE), lambda i: (i, 0, 0))
    bs_e = pl.BlockSpec((_BE, _E_DIM), lambda i: (i, 0))
    bs_n = pl.BlockSpec((_BA, _N_DIM), lambda i: (i, 0))
    bs_p = pl.BlockSpec((_BA, _E_DIM), lambda i: (i, 0))
    if last:
        out_shapes = [jax.ShapeDtypeStruct((nall, nnei, _E_DIM), jnp.float32)]
        out_specs = [pl.BlockSpec((_BA, nnei, _E_DIM), lambda i: (i, 0, 0))]
    else:
        out_shapes = [jax.ShapeDtypeStruct((nall * nnei, _E_DIM), jnp.float32)]
        out_specs = [bs_e]
    out_shapes += [jax.ShapeDtypeStruct((nall, _N_DIM), jnp.float32)]
    out_specs += [bs_n]
    if last:
        out_shapes += [jax.ShapeDtypeStruct((nall, _E_DIM), jnp.float32)] * 3
        out_specs += [bs_p] * 3
    else:
        out_shapes += [jax.ShapeDtypeStruct((nall, _E_DIM), jnp.float32)]
        out_specs += [bs_p]
    return pl.pallas_call(
        functools.partial(_tc_layer_body, last),
        grid=(nb,),
        in_specs=[
            bs_e, bs_e, bs_f, bs_n,
            _full((_N_DIM, _N_DIM)),
            _full((_E_DIM, _N_DIM)),
            _full((_E_DIM * _AXIS, _N_DIM)),
            _full((_E_DIM, _E_DIM)),
            _full((_N_DIM, _E_DIM)),
            _full((_N_DIM, _E_DIM)),
        ],
        out_specs=tuple(out_specs),
        out_shape=tuple(out_shapes),
    )(edge, pg, fld, node, wn1, wn2, wn3, w1e, w2e, w3n)


def _one_frame(coord, atype, nl, type_embedding, W_e_init, b_e_init, W_node,
               W_edge):
    nall, nnei = nl.shape
    nl = nl.astype(jnp.int32)
    idx_flat = nl.reshape(-1)

    coordp = jnp.concatenate(
        [coord.astype(jnp.float32),
         jnp.zeros((nall, 13), jnp.float32)], axis=1)
    cnb = _sc_gather(coordp, idx_flat)
    cnb3 = cnb.reshape(nall, nnei, 16)
    atype2 = atype.astype(jnp.int32).reshape(nall, 1)

    w3e = [W_edge[l][_E_DIM + _N_DIM:] for l in range(_NLAYERS)]
    be2 = b_e_init.reshape(1, _E_DIM)

    (fld, e1, e2, e3, edge, node, p) = _tc_init(
        cnb3, coordp, atype2, type_embedding, W_e_init, be2, w3e[0],
        nall, nnei)

    rot = None
    for l in range(_NLAYERS):
        pg = _sc_gather(p, idx_flat)
        wn = W_node[l]
        wn1, wn2 = wn[:_N_DIM], wn[_N_DIM:_N_DIM + _E_DIM]
        wn3 = (wn[_N_DIM + _E_DIM:]
               .reshape(_E_DIM, _AXIS, _N_DIM)
               .transpose(1, 0, 2)
               .reshape(_E_DIM * _AXIS, _N_DIM))
        w1e = W_edge[l][:_E_DIM]
        w2e = W_edge[l][_E_DIM:_E_DIM + _N_DIM]
        last = l == _NLAYERS - 1
        w3n = w3e[l + 1] if not last else w3e[l]
        res = _tc_layer(last, edge, pg, fld, node,
                        wn1, wn2, wn3, w1e, w2e, w3n, nall, nnei)
        if last:
            edge, node, r1, r2, r3 = res
            rot = jnp.stack([r1, r2, r3], axis=1)
        else:
            edge, node, p = res

    h2 = jnp.stack([e1, e2, e3], axis=-1)
    sw = fld[:, 0, :].reshape(nall, nnei)
    return node, edge, h2, rot, sw


def kernel(extended_coord, extended_atype, nlist, type_embedding, W_e_init,
           b_e_init, W_node, W_edge):
    nf = extended_coord.shape[0]
    outs = [
        _one_frame(extended_coord[f], extended_atype[f], nlist[f],
                   type_embedding, W_e_init, b_e_init, W_node, W_edge)
        for f in range(nf)
    ]
    return tuple(jnp.stack([o[i] for o in outs], axis=0) for i in range(5))
